# baseline (speedup 1.0000x reference)
import functools

import jax
import jax.numpy as jnp
from jax import lax
from jax.experimental import pallas as pl
from jax.experimental.pallas import tpu as pltpu
from jax.experimental.pallas import tpu_sc as plsc

L = 16
NC = 2
NS = 16
NW = NC * NS
NSLAB = 2
ROWS = 4096
COLS = 4096
WR = ROWS * NSLAB // NW // NSLAB
QMAX = 127.0
MAGIC = 1.5 * 2 ** 23

R1 = 8
CPS1 = WR // R1
NCH1 = NSLAB * CPS1
R2 = 4
CPS2 = WR // R2
NCH2 = NSLAB * CPS2
U = 8

_mesh = plsc.VectorSubcoreMesh(core_axis_name="c", subcore_axis_name="s")


@functools.partial(
    pl.kernel,
    mesh=_mesh,
    out_type=jax.ShapeDtypeStruct((NW * L,), jnp.float32),
    scratch_types=[
        pltpu.VMEM((R1, COLS), jnp.float32),
        pltpu.VMEM((R1, COLS), jnp.float32),
        pltpu.VMEM((L,), jnp.float32),
        pltpu.SemaphoreType.DMA,
        pltpu.SemaphoreType.DMA,
    ],
)
def _absmax_kernel(x_hbm, out_hbm, buf0, buf1, accb, sem0, sem1):
    wid = lax.axis_index("s") * NC + lax.axis_index("c")
    wrow = wid * WR
    bufs = (buf0, buf1)
    sems = (sem0, sem1)

    def src(c):
        return x_hbm.at[c // CPS1, pl.ds(wrow + (c % CPS1) * R1, R1), :]

    for b in range(2):
        pltpu.async_copy(src(b), bufs[b], sems[b])

    def body(c, acc):
        for b in range(2):
            cc = c + b
            buf = bufs[b]
            pltpu.make_async_copy(src(0), buf, sems[b]).wait()

            def row_body(r, a, buf=buf):
                def col_body(j, a2):
                    for u in range(U):
                        a2 = jnp.maximum(
                            a2, jnp.abs(buf[r, pl.ds((j * U + u) * L, L)])
                        )
                    return a2

                return lax.fori_loop(0, COLS // (L * U), col_body, a)

            acc = lax.fori_loop(0, R1, row_body, acc)
            nxt = cc + 2

            @pl.when(nxt < NCH1)
            def _(b=b, nxt=nxt):
                pltpu.async_copy(src(nxt), bufs[b], sems[b])

        return acc

    acc = pl.loop(0, NCH1, step=2, init_carry=jnp.zeros((L,), jnp.float32))(body)
    accb[...] = acc
    pltpu.sync_copy(accb, out_hbm.at[pl.ds(wid * L, L)])


@functools.partial(
    pl.kernel,
    mesh=_mesh,
    out_type=jax.ShapeDtypeStruct((NSLAB, ROWS, COLS), jnp.float32),
    scratch_types=[
        pltpu.VMEM((R2, COLS), jnp.float32),
        pltpu.VMEM((R2, COLS), jnp.float32),
        pltpu.VMEM((R2, COLS), jnp.float32),
        pltpu.VMEM((R2, COLS), jnp.float32),
        pltpu.VMEM((NW * L,), jnp.float32),
        pltpu.SemaphoreType.DMA,
        pltpu.SemaphoreType.DMA,
        pltpu.SemaphoreType.DMA,
        pltpu.SemaphoreType.DMA,
    ],
)
def _quant_kernel(x_hbm, pmax_hbm, out_hbm, in0, in1, ob0, ob1, pbuf,
                  isem0, isem1, osem0, osem1):
    wid = lax.axis_index("s") * NC + lax.axis_index("c")
    wrow = wid * WR
    ibufs = (in0, in1)
    isems = (isem0, isem1)
    obufs = (ob0, ob1)
    osems = (osem0, osem1)

    def src(c):
        return x_hbm.at[c // CPS2, pl.ds(wrow + (c % CPS2) * R2, R2), :]

    def dst(c):
        return out_hbm.at[c // CPS2, pl.ds(wrow + (c % CPS2) * R2, R2), :]

    pltpu.sync_copy(pmax_hbm, pbuf)
    v = pbuf[pl.ds(0, L)]
    for i in range(1, NW):
        v = jnp.maximum(v, pbuf[pl.ds(i * L, L)])
    absmax = v[0]
    for i in range(1, L):
        absmax = jnp.maximum(absmax, v[i])
    scale = jnp.full((L,), absmax, jnp.float32) * (1.0 / QMAX)
    yi = 0x7EB53567 - lax.bitcast_convert_type(scale, jnp.int32)
    inv = lax.bitcast_convert_type(yi, jnp.float32)
    for _ in range(4):
        inv = inv * (2.0 - scale * inv)

    for b in range(2):
        pltpu.async_copy(src(b), ibufs[b], isems[b])

    def body(c, carry):
        for b in range(2):
            cc = c + b
            ibuf, obuf = ibufs[b], obufs[b]
            pltpu.make_async_copy(src(0), ibuf, isems[b]).wait()

            @pl.when(cc >= 2)
            def _(b=b):
                pltpu.make_async_copy(obufs[b], dst(0), osems[b]).wait()

            def row_body(r, d, ibuf=ibuf, obuf=obuf):
                def col_body(j, d2):
                    for u in range(U):
                        off = (j * U + u) * L
                        t = ibuf[r, pl.ds(off, L)] * inv
                        q = (t + MAGIC) - MAGIC
                        obuf[r, pl.ds(off, L)] = q * scale
                    return d2

                return lax.fori_loop(0, COLS // (L * U), col_body, d)

            lax.fori_loop(0, R2, row_body, 0)
            pltpu.async_copy(obuf, dst(cc), osems[b])
            nxt = cc + 2

            @pl.when(nxt < NCH2)
            def _(b=b, nxt=nxt):
                pltpu.async_copy(src(nxt), ibufs[b], isems[b])

        return carry

    pl.loop(0, NCH2, step=2, init_carry=0)(body)
    for b in range(2):
        pltpu.make_async_copy(obufs[b], dst(0), osems[b]).wait()


def kernel(tensor):
    pmax = _absmax_kernel(tensor)
    return _quant_kernel(tensor, pmax)


# speedup vs baseline: 1.0030x; 1.0030x over previous
import functools

import jax
import jax.numpy as jnp
from jax import lax
from jax.experimental import pallas as pl
from jax.experimental.pallas import tpu as pltpu
from jax.experimental.pallas import tpu_sc as plsc

L = 16
NC = 2
NS = 16
NW = NC * NS
NSLAB = 2
ROWS = 4096
COLS = 4096
WR = ROWS * NSLAB // NW // NSLAB
QMAX = 127.0
MAGIC = 1.5 * 2 ** 23

R1 = 8
CPS1 = WR // R1
NCH1 = NSLAB * CPS1
R2 = 8
C2 = COLS // 2
CPS2 = (WR // R2) * 2
NCH2 = NSLAB * CPS2
U = 8

_mesh = plsc.VectorSubcoreMesh(core_axis_name="c", subcore_axis_name="s")


@functools.partial(
    pl.kernel,
    mesh=_mesh,
    out_type=jax.ShapeDtypeStruct((NW * L,), jnp.float32),
    scratch_types=[
        pltpu.VMEM((R1, COLS), jnp.float32),
        pltpu.VMEM((R1, COLS), jnp.float32),
        pltpu.VMEM((L,), jnp.float32),
        pltpu.SemaphoreType.DMA,
        pltpu.SemaphoreType.DMA,
    ],
)
def _absmax_kernel(x_hbm, out_hbm, buf0, buf1, accb, sem0, sem1):
    wid = lax.axis_index("s") * NC + lax.axis_index("c")
    wrow = wid * WR
    bufs = (buf0, buf1)
    sems = (sem0, sem1)

    def src(c):
        return x_hbm.at[c // CPS1, pl.ds(wrow + (c % CPS1) * R1, R1), :]

    for b in range(2):
        pltpu.async_copy(src(b), bufs[b], sems[b])

    def body(c, acc):
        for b in range(2):
            cc = c + b
            buf = bufs[b]
            pltpu.make_async_copy(src(0), buf, sems[b]).wait()

            def row_body(r, a, buf=buf):
                def col_body(j, a2):
                    for u in range(U):
                        a2 = jnp.maximum(
                            a2, jnp.abs(buf[r, pl.ds((j * U + u) * L, L)])
                        )
                    return a2

                return lax.fori_loop(0, COLS // (L * U), col_body, a)

            acc = lax.fori_loop(0, R1, row_body, acc)
            nxt = cc + 2

            @pl.when(nxt < NCH1)
            def _(b=b, nxt=nxt):
                pltpu.async_copy(src(nxt), bufs[b], sems[b])

        return acc

    acc = pl.loop(0, NCH1, step=2, init_carry=jnp.zeros((L,), jnp.float32))(body)
    accb[...] = acc
    pltpu.sync_copy(accb, out_hbm.at[pl.ds(wid * L, L)])


@functools.partial(
    pl.kernel,
    mesh=_mesh,
    out_type=jax.ShapeDtypeStruct((NSLAB, ROWS, COLS), jnp.float32),
    scratch_types=[
        pltpu.VMEM((R2, C2), jnp.float32),
        pltpu.VMEM((R2, C2), jnp.float32),
        pltpu.VMEM((R2, C2), jnp.float32),
        pltpu.VMEM((R2, C2), jnp.float32),
        pltpu.VMEM((NW * L,), jnp.float32),
        pltpu.SemaphoreType.DMA,
        pltpu.SemaphoreType.DMA,
        pltpu.SemaphoreType.DMA,
        pltpu.SemaphoreType.DMA,
    ],
)
def _quant_kernel(x_hbm, pmax_hbm, out_hbm, in0, in1, ob0, ob1, pbuf,
                  isem0, isem1, osem0, osem1):
    wid = lax.axis_index("s") * NC + lax.axis_index("c")
    wrow = wid * WR
    ibufs = (in0, in1)
    isems = (isem0, isem1)
    obufs = (ob0, ob1)
    osems = (osem0, osem1)

    def _slice(c):
        s = c // CPS2
        k = c % CPS2
        r0 = wrow + (k // 2) * R2
        c0 = (k % 2) * C2
        return s, r0, c0

    def src(c):
        s, r0, c0 = _slice(c)
        return x_hbm.at[s, pl.ds(r0, R2), pl.ds(c0, C2)]

    def dst(c):
        s, r0, c0 = _slice(c)
        return out_hbm.at[s, pl.ds(r0, R2), pl.ds(c0, C2)]

    pltpu.sync_copy(pmax_hbm, pbuf)
    v = pbuf[pl.ds(0, L)]
    for i in range(1, NW):
        v = jnp.maximum(v, pbuf[pl.ds(i * L, L)])
    absmax = v[0]
    for i in range(1, L):
        absmax = jnp.maximum(absmax, v[i])
    scale = jnp.full((L,), absmax, jnp.float32) * (1.0 / QMAX)
    yi = 0x7EB53567 - lax.bitcast_convert_type(scale, jnp.int32)
    inv = lax.bitcast_convert_type(yi, jnp.float32)
    for _ in range(4):
        inv = inv * (2.0 - scale * inv)

    for b in range(2):
        pltpu.async_copy(src(b), ibufs[b], isems[b])

    def body(c, carry):
        for b in range(2):
            cc = c + b
            ibuf, obuf = ibufs[b], obufs[b]
            pltpu.make_async_copy(src(0), ibuf, isems[b]).wait()

            @pl.when(cc >= 2)
            def _(b=b):
                pltpu.make_async_copy(obufs[b], dst(0), osems[b]).wait()

            def row_body(r, d, ibuf=ibuf, obuf=obuf):
                def col_body(j, d2):
                    for u in range(U):
                        off = (j * U + u) * L
                        t = ibuf[r, pl.ds(off, L)] * inv
                        q = (t + MAGIC) - MAGIC
                        obuf[r, pl.ds(off, L)] = q * scale
                    return d2

                return lax.fori_loop(0, C2 // (L * U), col_body, d)

            lax.fori_loop(0, R2, row_body, 0)
            pltpu.async_copy(obuf, dst(cc), osems[b])
            nxt = cc + 2

            @pl.when(nxt < NCH2)
            def _(b=b, nxt=nxt):
                pltpu.async_copy(src(nxt), ibufs[b], isems[b])

        return carry

    pl.loop(0, NCH2, step=2, init_carry=0)(body)
    for b in range(2):
        pltpu.make_async_copy(obufs[b], dst(0), osems[b]).wait()


def kernel(tensor):
    pmax = _absmax_kernel(tensor)
    return _quant_kernel(tensor, pmax)


# speedup vs baseline: 3.6426x; 3.6316x over previous
import functools

import jax
import jax.numpy as jnp
from jax import lax
from jax.experimental import pallas as pl
from jax.experimental.pallas import tpu as pltpu
from jax.experimental.pallas import tpu_sc as plsc

L = 16
NC = 2
NS = 16
NW = NC * NS
NSLAB = 2
ROWS = 4096
COLS = 4096
WR = ROWS * NSLAB // NW // NSLAB
QMAX = 127.0
MAGIC = 1.5 * 2 ** 23

R1 = 8
CPS1 = WR // R1
NCH1 = NSLAB * CPS1
R2 = 8
C2 = COLS // 2
CPS2 = (WR // R2) * 2
NCH2 = NSLAB * CPS2
U = 8

_mesh = plsc.VectorSubcoreMesh(core_axis_name="c", subcore_axis_name="s")


@functools.partial(
    pl.kernel,
    mesh=_mesh,
    out_type=jax.ShapeDtypeStruct((NW * L,), jnp.float32),
    scratch_types=[
        pltpu.VMEM((R1, COLS), jnp.float32),
        pltpu.VMEM((R1, COLS), jnp.float32),
        pltpu.VMEM((L,), jnp.float32),
        pltpu.SemaphoreType.DMA,
        pltpu.SemaphoreType.DMA,
    ],
)
def _absmax_kernel(x_hbm, out_hbm, buf0, buf1, accb, sem0, sem1):
    wid = lax.axis_index("s") * NC + lax.axis_index("c")
    wrow = wid * WR
    bufs = (buf0, buf1)
    sems = (sem0, sem1)

    def src(c):
        return x_hbm.at[c // CPS1, pl.ds(wrow + (c % CPS1) * R1, R1), :]

    for b in range(2):
        pltpu.async_copy(src(b), bufs[b], sems[b])

    def body(c, acc):
        for b in range(2):
            cc = c + b
            buf = bufs[b]
            pltpu.make_async_copy(src(0), buf, sems[b]).wait()

            def row_body(r, a, buf=buf):
                def col_body(j, a2):
                    for u in range(U):
                        a2 = jnp.maximum(
                            a2, jnp.abs(buf[r, pl.ds((j * U + u) * L, L)])
                        )
                    return a2

                return lax.fori_loop(0, COLS // (L * U), col_body, a)

            acc = lax.fori_loop(0, R1, row_body, acc)
            nxt = cc + 2

            @pl.when(nxt < NCH1)
            def _(b=b, nxt=nxt):
                pltpu.async_copy(src(nxt), bufs[b], sems[b])

        return acc

    acc = pl.loop(0, NCH1, step=2, init_carry=jnp.zeros((L,), jnp.float32))(body)
    accb[...] = acc
    pltpu.sync_copy(accb, out_hbm.at[pl.ds(wid * L, L)])


@functools.partial(
    pl.kernel,
    mesh=_mesh,
    out_type=jax.ShapeDtypeStruct((NSLAB, ROWS, COLS), jnp.float32),
    scratch_types=[
        pltpu.VMEM((R2, C2), jnp.float32),
        pltpu.VMEM((R2, C2), jnp.float32),
        pltpu.VMEM((R2, C2), jnp.float32),
        pltpu.VMEM((R2, C2), jnp.float32),
        pltpu.VMEM((NW * L,), jnp.float32),
        pltpu.SemaphoreType.DMA,
        pltpu.SemaphoreType.DMA,
        pltpu.SemaphoreType.DMA,
        pltpu.SemaphoreType.DMA,
    ],
)
def _quant_kernel(x_hbm, pmax_hbm, out_hbm, in0, in1, ob0, ob1, pbuf,
                  isem0, isem1, osem0, osem1):
    wid = lax.axis_index("s") * NC + lax.axis_index("c")
    wrow = wid * WR
    ibufs = (in0, in1)
    isems = (isem0, isem1)
    obufs = (ob0, ob1)
    osems = (osem0, osem1)

    def _slice(c):
        s = c // CPS2
        k = c % CPS2
        r0 = wrow + (k // 2) * R2
        c0 = (k % 2) * C2
        return s, r0, c0

    def src(c):
        s, r0, c0 = _slice(c)
        return x_hbm.at[s, pl.ds(r0, R2), pl.ds(c0, C2)]

    def dst(c):
        s, r0, c0 = _slice(c)
        return out_hbm.at[s, pl.ds(r0, R2), pl.ds(c0, C2)]

    pltpu.sync_copy(pmax_hbm, pbuf)
    v = pbuf[pl.ds(0, L)]
    for i in range(1, NW):
        v = jnp.maximum(v, pbuf[pl.ds(i * L, L)])
    absmax = v[0]
    for i in range(1, L):
        absmax = jnp.maximum(absmax, v[i])
    scale = jnp.full((L,), absmax, jnp.float32) * (1.0 / QMAX)
    yi = 0x7EB53567 - lax.bitcast_convert_type(scale, jnp.int32)
    inv = lax.bitcast_convert_type(yi, jnp.float32)
    for _ in range(4):
        inv = inv * (2.0 - scale * inv)

    for b in range(2):
        pltpu.async_copy(src(b), ibufs[b], isems[b])

    def body(c, carry):
        for b in range(2):
            cc = c + b
            ibuf, obuf = ibufs[b], obufs[b]
            pltpu.make_async_copy(src(0), ibuf, isems[b]).wait()

            @pl.when(cc >= 2)
            def _(b=b):
                pltpu.make_async_copy(obufs[b], dst(0), osems[b]).wait()

            for r in range(R2):
                @plsc.parallel_loop(0, C2 // L, unroll=U)
                def _(vi, r=r, ibuf=ibuf, obuf=obuf):
                    off = vi * L
                    t = ibuf[r, pl.ds(off, L)] * inv
                    q = (t + MAGIC) - MAGIC
                    obuf[r, pl.ds(off, L)] = q * scale
            pltpu.async_copy(obuf, dst(cc), osems[b])
            nxt = cc + 2

            @pl.when(nxt < NCH2)
            def _(b=b, nxt=nxt):
                pltpu.async_copy(src(nxt), ibufs[b], isems[b])

        return carry

    pl.loop(0, NCH2, step=2, init_carry=0)(body)
    for b in range(2):
        pltpu.make_async_copy(obufs[b], dst(0), osems[b]).wait()


def kernel(tensor):
    pmax = _absmax_kernel(tensor)
    return _quant_kernel(tensor, pmax)
